# Initial kernel scaffold; baseline (speedup 1.0000x reference)
import jax
import jax.numpy as jnp
from jax.experimental import pallas as pl


def kernel(emb, logits, labels, centers):
    raise NotImplementedError("write your pallas kernel here")


# speedup vs baseline: 9.2058x; 9.2058x over previous
import functools

import jax
import jax.numpy as jnp
from jax import lax
from jax.experimental import pallas as pl
from jax.experimental.pallas import tpu as pltpu
from jax.experimental.pallas import tpu_sc as plsc

_NUM_CLASSES = 1000
_FEAT = 512
_B = 1024
_TEMP = 0.1
_ALPHA = 0.1
_BETA = 0.01
_MARGIN = 0.01
_GAMMA = 0.9
_K_HARD = 0.4
_EPS = 1e-16
_SM = _MARGIN / _TEMP
_NEG_FILL = -1e30

_C_PAD = 1024
_NC, _NS = 2, 16
_NW = _NC * _NS
_RPW = _B // _NW
_CROWS = _C_PAD // _NS

_ROW_T = 128
_NT = _B // _ROW_T
_BS_ITERS = 24


def _l2norm_body(x_ref, o_ref):
    x = x_ref[...]
    nrm = jnp.sqrt(jnp.sum(x * x, axis=1, keepdims=True))
    o_ref[...] = x / jnp.maximum(nrm, 1e-12)


def _l2norm_tc(emb):
    return pl.pallas_call(
        _l2norm_body,
        grid=(_NT,),
        in_specs=[pl.BlockSpec((_ROW_T, _FEAT), lambda i: (i, 0))],
        out_specs=pl.BlockSpec((_ROW_T, _FEAT), lambda i: (i, 0)),
        out_shape=jax.ShapeDtypeStruct((_B, _FEAT), jnp.float32),
    )(emb)


_FPW = _FEAT // _NW


def _sc_body(en_hbm, labels_hbm, sums_hbm, lab_v, rows_v, acc_v):
    c = lax.axis_index("c")
    s = lax.axis_index("s")
    wid = s * _NC + c
    fs = wid * _FPW

    pltpu.sync_copy(labels_hbm, lab_v)
    pltpu.sync_copy(en_hbm.at[:, pl.ds(fs, _FPW)], rows_v)

    z16 = jnp.zeros((_FPW,), jnp.float32)

    def _zero(r, _):
        acc_v[r, :] = z16
        return 0
    lax.fori_loop(0, _C_PAD, _zero, 0)

    def _accum(rc, _):
        base = rc * 16
        lab16 = lab_v[pl.ds(base, 16)]
        for j in range(16):
            plsc.addupdate(acc_v.at[lab16[j]], rows_v[base + j, :])
        return 0
    lax.fori_loop(0, _B // 16, _accum, 0)

    pltpu.sync_copy(acc_v, sums_hbm.at[:, pl.ds(fs, _FPW)])


def _center_partials(en, labels):
    f = pl.kernel(
        _sc_body,
        out_type=jax.ShapeDtypeStruct((_C_PAD, _FEAT), jnp.float32),
        mesh=plsc.VectorSubcoreMesh(
            core_axis_name="c", subcore_axis_name="s",
            num_cores=_NC, num_subcores=_NS),
        scratch_types=[
            pltpu.VMEM((_B,), jnp.int32),
            pltpu.VMEM((_B, _FPW), jnp.float32),
            pltpu.VMEM((_C_PAD, _FPW), jnp.float32),
        ],
        compiler_params=pltpu.CompilerParams(use_tc_tiling_on_sc=False),
    )
    return f(en, labels)


def _main_body(en_t_ref, en_ref, lg_ref, lrow_ref, lcol_ref, ctr_ref,
               psums_ref, out_ref, cneg_ref, sacc_ref):
    i = pl.program_id(0)

    @pl.when(i == 0)
    def _init():
        for t in range(5):
            sacc_ref[t] = 0.0

    inv_t = jnp.float32(1.0 / _TEMP)
    lrow = lrow_ref[...]
    lcol = lcol_ref[...]

    en_t = en_t_ref[...]
    cs = lax.dot_general(en_t, en_ref[...], (((1,), (1,)), ((), ())),
                         preferred_element_type=jnp.float32) * inv_t
    same = lcol == lrow
    rowid = i * _ROW_T + lax.broadcasted_iota(jnp.int32, (_ROW_T, _B), 0)
    colid = lax.broadcasted_iota(jnp.int32, (_ROW_T, _B), 1)
    pos = same & (rowid != colid)
    neg = ~same
    sacc_ref[0] += jnp.sum(pos.astype(jnp.float32))
    sacc_ref[1] += jnp.sum(jnp.where(pos, jnp.maximum(1.0 - cs, 0.0), 0.0))
    sacc_ref[2] += jnp.sum(neg.astype(jnp.float32))
    sacc_ref[3] += jnp.sum(jnp.where(neg, jnp.maximum(cs - _SM, 0.0), 0.0))
    cneg_ref[pl.ds(i * _ROW_T, _ROW_T), :] = jnp.where(neg, cs, _NEG_FILL)

    lg = lg_ref[...] * inv_t
    mx = jnp.max(lg, axis=1, keepdims=True)
    lse = jnp.log(jnp.sum(jnp.exp(lg - mx), axis=1, keepdims=True)) + mx
    cid = lax.broadcasted_iota(jnp.int32, (_ROW_T, _NUM_CLASSES), 1)
    lab = jnp.sum(jnp.where(cid == lcol, lg, 0.0), axis=1, keepdims=True)
    sacc_ref[4] += jnp.sum(lse - lab)

    @pl.when(i == _NT - 1)
    def _finish():
        def count_gt(t):
            def body(j, acc):
                blk = cneg_ref[pl.ds(j * _ROW_T, _ROW_T), :]
                return acc + jnp.sum((blk > t).astype(jnp.float32))
            return lax.fori_loop(0, _NT, body, jnp.float32(0.0))

        m = count_gt(jnp.float32(_SM))
        kf = jnp.maximum(1.0, jnp.floor(jnp.float32(_K_HARD) * m))

        def bs(_, carry):
            lo, hi = carry
            mid = 0.5 * (lo + hi)
            c = count_gt(mid)
            take = c >= kf
            return (jnp.where(take, mid, lo), jnp.where(take, hi, mid))
        lo, _hi = lax.fori_loop(
            0, _BS_ITERS, bs, (jnp.float32(_SM), jnp.float32(10.5)))

        def fs(j, carry):
            sm_, cn_ = carry
            blk = cneg_ref[pl.ds(j * _ROW_T, _ROW_T), :]
            gt = blk > lo
            return (sm_ + jnp.sum(jnp.where(gt, blk, 0.0)),
                    cn_ + jnp.sum(gt.astype(jnp.float32)))
        s_gt, c_gt = lax.fori_loop(
            0, _NT, fs, (jnp.float32(0.0), jnp.float32(0.0)))
        topk_sum = s_gt - (c_gt - kf) * lo
        loss_neg_hard = topk_sum / kf - _SM
        loss_neg_fb = sacc_ref[3] / sacc_ref[2]
        loss_neg = jnp.where(m > 0.0, loss_neg_hard, loss_neg_fb)
        loss_pos = sacc_ref[1] / sacc_ref[0]
        loss_cos = jnp.maximum(loss_pos + loss_neg, _EPS)
        loss_ce = sacc_ref[4] / _B

        def ct(j, acc):
            sl = pl.ds(j * _ROW_T, _ROW_T)
            sm_ = psums_ref[sl, :]
            cls = j * _ROW_T + lax.broadcasted_iota(
                jnp.int32, (_ROW_T, _B), 0)
            cnt = jnp.sum((lrow == cls).astype(jnp.float32), axis=1,
                          keepdims=True)
            ctr = ctr_ref[sl, :]
            newc = sm_ / (cnt + _EPS)
            upd = jnp.where(cnt > 0.0, _GAMMA * ctr + (1.0 - _GAMMA) * newc,
                            ctr)
            nrm = jnp.maximum(
                jnp.sqrt(jnp.sum(upd * upd, axis=1, keepdims=True)), 1e-12)
            dot = jnp.sum(upd * sm_, axis=1, keepdims=True) / nrm
            return acc + jnp.sum(dot)
        tot = lax.fori_loop(0, _NT, ct, jnp.float32(0.0))
        loss_center = jnp.maximum(1.0 - tot / (_B * _TEMP), _EPS)

        total = loss_ce + _ALPHA * loss_cos + _BETA * loss_center
        out_ref[...] = jnp.broadcast_to(total, (1, 1))


def _main(en, logits, lrow, lcol, centers_p, psums):
    return pl.pallas_call(
        _main_body,
        grid=(_NT,),
        in_specs=[
            pl.BlockSpec((_ROW_T, _FEAT), lambda i: (i, 0)),
            pl.BlockSpec((_B, _FEAT), lambda i: (0, 0)),
            pl.BlockSpec((_ROW_T, _NUM_CLASSES), lambda i: (i, 0)),
            pl.BlockSpec((1, _B), lambda i: (0, 0)),
            pl.BlockSpec((_ROW_T, 1), lambda i: (i, 0)),
            pl.BlockSpec((_C_PAD, _FEAT), lambda i: (0, 0)),
            pl.BlockSpec((_C_PAD, _FEAT), lambda i: (0, 0)),
        ],
        out_specs=pl.BlockSpec((1, 1), lambda i: (0, 0)),
        out_shape=jax.ShapeDtypeStruct((1, 1), jnp.float32),
        scratch_shapes=[
            pltpu.VMEM((_B, _B), jnp.float32),
            pltpu.SMEM((8,), jnp.float32),
        ],
    )(en, en, logits, lrow, lcol, centers_p, psums)


def kernel(emb, logits, labels, centers):
    en = _l2norm_tc(emb)
    psums = _center_partials(en, labels)
    lrow = labels.reshape(1, _B)
    lcol = labels.reshape(_B, 1)
    centers_p = jnp.zeros((_C_PAD, _FEAT), jnp.float32).at[:_NUM_CLASSES].set(
        centers)
    out = _main(en, logits, lrow, lcol, centers_p, psums)
    return out[0, 0]


# speedup vs baseline: 17.6935x; 1.9220x over previous
import functools

import jax
import jax.numpy as jnp
from jax import lax
from jax.experimental import pallas as pl
from jax.experimental.pallas import tpu as pltpu
from jax.experimental.pallas import tpu_sc as plsc

_NUM_CLASSES = 1000
_FEAT = 512
_B = 1024
_TEMP = 0.1
_ALPHA = 0.1
_BETA = 0.01
_MARGIN = 0.01
_GAMMA = 0.9
_K_HARD = 0.4
_EPS = 1e-16
_SM = _MARGIN / _TEMP
_NEG_FILL = -1e30

_C_PAD = 1024
_NC, _NS = 2, 16
_NW = _NC * _NS
_RPW = _B // _NW
_CROWS = _C_PAD // _NS

_ROW_T = 128
_NT = _B // _ROW_T
_BS_ITERS = 16


def _l2norm_body(x_ref, o_ref):
    x = x_ref[...]
    nrm = jnp.sqrt(jnp.sum(x * x, axis=1, keepdims=True))
    o_ref[...] = x / jnp.maximum(nrm, 1e-12)


def _l2norm_tc(emb):
    return pl.pallas_call(
        _l2norm_body,
        grid=(_NT,),
        in_specs=[pl.BlockSpec((_ROW_T, _FEAT), lambda i: (i, 0))],
        out_specs=pl.BlockSpec((_ROW_T, _FEAT), lambda i: (i, 0)),
        out_shape=jax.ShapeDtypeStruct((_B, _FEAT), jnp.float32),
    )(emb)


_FPW = _FEAT // _NW


def _sc_body(en_hbm, labels_hbm, sums_hbm, lab_v, rows_v, acc_v):
    c = lax.axis_index("c")
    s = lax.axis_index("s")
    wid = s * _NC + c
    fs = wid * _FPW

    pltpu.sync_copy(labels_hbm, lab_v)
    pltpu.sync_copy(en_hbm.at[:, pl.ds(fs, _FPW)], rows_v)

    z16 = jnp.zeros((_FPW,), jnp.float32)

    def _zero(r, _):
        acc_v[r, :] = z16
        return 0
    lax.fori_loop(0, _C_PAD, _zero, 0)

    def _accum(rc, _):
        base = rc * 16
        lab16 = lab_v[pl.ds(base, 16)]
        for j in range(16):
            plsc.addupdate(acc_v.at[lab16[j]], rows_v[base + j, :])
        return 0
    lax.fori_loop(0, _B // 16, _accum, 0)

    pltpu.sync_copy(acc_v, sums_hbm.at[:, pl.ds(fs, _FPW)])


def _center_partials(en, labels):
    f = pl.kernel(
        _sc_body,
        out_type=jax.ShapeDtypeStruct((_C_PAD, _FEAT), jnp.float32),
        mesh=plsc.VectorSubcoreMesh(
            core_axis_name="c", subcore_axis_name="s",
            num_cores=_NC, num_subcores=_NS),
        scratch_types=[
            pltpu.VMEM((_B,), jnp.int32),
            pltpu.VMEM((_B, _FPW), jnp.float32),
            pltpu.VMEM((_C_PAD, _FPW), jnp.float32),
        ],
        compiler_params=pltpu.CompilerParams(use_tc_tiling_on_sc=False),
    )
    return f(en, labels)


def _main_body(en_t_ref, en_ref, lg_ref, lrow_ref, lcol_ref,
               out_ref, cneg_ref, sacc_ref):
    i = pl.program_id(0)

    @pl.when(i == 0)
    def _init():
        for t in range(6):
            sacc_ref[t] = 0.0

    inv_t = jnp.float32(1.0 / _TEMP)
    lrow = lrow_ref[...]
    lcol = lcol_ref[...]

    en_t = en_t_ref[...]
    cs = lax.dot_general(en_t, en_ref[...], (((1,), (1,)), ((), ())),
                         preferred_element_type=jnp.float32) * inv_t
    same = lcol == lrow
    rowid = i * _ROW_T + lax.broadcasted_iota(jnp.int32, (_ROW_T, _B), 0)
    colid = lax.broadcasted_iota(jnp.int32, (_ROW_T, _B), 1)
    upper = colid > rowid
    pos = same & upper
    neg = (~same) & upper
    sacc_ref[0] += jnp.sum(pos.astype(jnp.float32))
    sacc_ref[1] += jnp.sum(jnp.where(pos, jnp.maximum(1.0 - cs, 0.0), 0.0))
    sacc_ref[2] += jnp.sum(neg.astype(jnp.float32))
    sacc_ref[3] += jnp.sum(jnp.where(neg, jnp.maximum(cs - _SM, 0.0), 0.0))
    sacc_ref[5] += jnp.sum((neg & (cs > _SM)).astype(jnp.float32))
    cneg_ref[pl.ds(i * _ROW_T, _ROW_T), :] = jnp.where(neg, cs, _NEG_FILL)

    lg = lg_ref[...] * inv_t
    mx = jnp.max(lg, axis=1, keepdims=True)
    lse = jnp.log(jnp.sum(jnp.exp(lg - mx), axis=1, keepdims=True)) + mx
    cid = lax.broadcasted_iota(jnp.int32, (_ROW_T, _NUM_CLASSES), 1)
    lab = jnp.sum(jnp.where(cid == lcol, lg, 0.0), axis=1, keepdims=True)
    sacc_ref[4] += jnp.sum(lse - lab)

    @pl.when(i == _NT - 1)
    def _tail():
        def count_gt(t):
            acc = jnp.float32(0.0)
            for bi in range(_NT):
                for bj in range(bi, _NT):
                    blk = cneg_ref[bi * _ROW_T:(bi + 1) * _ROW_T,
                                   bj * _ROW_T:(bj + 1) * _ROW_T]
                    acc += jnp.sum((blk > t).astype(jnp.float32))
            return acc

        m = 2.0 * sacc_ref[5]
        kf = jnp.maximum(1.0, jnp.floor(jnp.float32(_K_HARD) * m))

        def bs(_, carry):
            lo, hi = carry
            mid = 0.5 * (lo + hi)
            c = 2.0 * count_gt(mid)
            take = c >= kf
            return (jnp.where(take, mid, lo), jnp.where(take, hi, mid))
        lo, _hi = lax.fori_loop(
            0, _BS_ITERS, bs, (jnp.float32(_SM), jnp.float32(10.5)))

        s_gt = jnp.float32(0.0)
        c_gt = jnp.float32(0.0)
        for bi in range(_NT):
            for bj in range(bi, _NT):
                blk = cneg_ref[bi * _ROW_T:(bi + 1) * _ROW_T,
                               bj * _ROW_T:(bj + 1) * _ROW_T]
                gt = blk > lo
                s_gt += jnp.sum(jnp.where(gt, blk, 0.0))
                c_gt += jnp.sum(gt.astype(jnp.float32))
        topk_sum = 2.0 * s_gt - (2.0 * c_gt - kf) * lo
        loss_neg_hard = topk_sum / kf - _SM
        loss_neg_fb = sacc_ref[3] / sacc_ref[2]
        loss_neg = jnp.where(m > 0.0, loss_neg_hard, loss_neg_fb)
        loss_pos = sacc_ref[1] / sacc_ref[0]
        loss_cos = jnp.maximum(loss_pos + loss_neg, _EPS)
        loss_ce = sacc_ref[4] / _B
        partial = loss_ce + _ALPHA * loss_cos
        out_ref[...] = jnp.broadcast_to(partial, (1, 1))


def _main(en, logits, lrow, lcol):
    return pl.pallas_call(
        _main_body,
        grid=(_NT,),
        in_specs=[
            pl.BlockSpec((_ROW_T, _FEAT), lambda i: (i, 0)),
            pl.BlockSpec((_B, _FEAT), lambda i: (0, 0)),
            pl.BlockSpec((_ROW_T, _NUM_CLASSES), lambda i: (i, 0)),
            pl.BlockSpec((1, _B), lambda i: (0, 0)),
            pl.BlockSpec((_ROW_T, 1), lambda i: (i, 0)),
        ],
        out_specs=pl.BlockSpec((1, 1), lambda i: (0, 0)),
        out_shape=jax.ShapeDtypeStruct((1, 1), jnp.float32),
        scratch_shapes=[
            pltpu.VMEM((_B, _B), jnp.float32),
            pltpu.SMEM((8,), jnp.float32),
        ],
    )(en, en, logits, lrow, lcol)


def _finish_body(part_ref, lrow_ref, ctr_ref, psums_ref, out_ref):
    lrow = lrow_ref[...]

    def ct(j, acc):
        sl = pl.ds(j * _ROW_T, _ROW_T)
        sm_ = psums_ref[sl, :]
        cls = j * _ROW_T + lax.broadcasted_iota(jnp.int32, (_ROW_T, _B), 0)
        cnt = jnp.sum((lrow == cls).astype(jnp.float32), axis=1,
                      keepdims=True)
        ctr = ctr_ref[sl, :]
        newc = sm_ / (cnt + _EPS)
        upd = jnp.where(cnt > 0.0, _GAMMA * ctr + (1.0 - _GAMMA) * newc, ctr)
        nrm = jnp.maximum(
            jnp.sqrt(jnp.sum(upd * upd, axis=1, keepdims=True)), 1e-12)
        dot = jnp.sum(upd * sm_, axis=1, keepdims=True) / nrm
        return acc + jnp.sum(dot)
    tot = lax.fori_loop(0, _NT, ct, jnp.float32(0.0))
    loss_center = jnp.maximum(1.0 - tot / (_B * _TEMP), _EPS)
    out_ref[...] = part_ref[...] + _BETA * loss_center


def _finish(partial, lrow, centers_p, psums):
    return pl.pallas_call(
        _finish_body,
        out_shape=jax.ShapeDtypeStruct((1, 1), jnp.float32),
    )(partial, lrow, centers_p, psums)


def kernel(emb, logits, labels, centers):
    en = _l2norm_tc(emb)
    psums = _center_partials(en, labels)
    lrow = labels.reshape(1, _B)
    lcol = labels.reshape(_B, 1)
    centers_p = jnp.zeros((_C_PAD, _FEAT), jnp.float32).at[:_NUM_CLASSES].set(
        centers)
    partial = _main(en, logits, lrow, lcol)
    out = _finish(partial, lrow, centers_p, psums)
    return out[0, 0]
